# baseline (speedup 1.0000x reference)
import functools
import math

import jax
import jax.numpy as jnp
from jax import lax
from jax.experimental import pallas as pl
from jax.experimental.pallas import tpu as pltpu
from jax.experimental.pallas import tpu_sc as plsc

_NC = 2
_NS = 16
_C = 128
_G = 64
_BN = 1000
_PAD_ROWS = 16


def _sc_aggregate(h, src2d, dst2d):
    N, D = h.shape
    nchunks = src2d.shape[0]
    NW = _NC * _NS
    cw = nchunks // NW
    rows_per_tile = (N // _NS) // 8 * 8
    rem_rows = N - rows_per_tile * _NS
    n_acc = N + _PAD_ROWS
    mesh = plsc.VectorSubcoreMesh(core_axis_name="c", subcore_axis_name="s",
                                  num_cores=_NC, num_subcores=_NS)

    @functools.partial(
        pl.kernel,
        out_type=jax.ShapeDtypeStruct((_NC, N, D), jnp.float32),
        mesh=mesh,
        scratch_types=[
            [pltpu.VMEM((_C,), jnp.int32) for _ in range(3)],
            [pltpu.VMEM((_C,), jnp.int32) for _ in range(3)],
            [pltpu.VMEM((_C,), jnp.int32) for _ in range(3)],
            [pltpu.VMEM((_C, D), jnp.float32) for _ in range(3)],
            pltpu.VMEM_SHARED((n_acc, D), jnp.float32),
            [pltpu.SemaphoreType.DMA for _ in range(3)],
            [pltpu.SemaphoreType.DMA for _ in range(3)],
            [pltpu.SemaphoreType.DMA for _ in range(3)],
        ],
    )
    def agg_kernel(h_hbm, src_hbm, dst_hbm, out_hbm, src_b, dst_b, dstv,
                   rows, agg_sh, isems, gsems, ssems):
        cid = lax.axis_index("c")
        sid = lax.axis_index("s")
        w = sid * _NC + cid
        c0 = w * cw

        for b in range(2):
            pltpu.sync_copy(src_hbm.at[c0 + b], src_b[b])
            pltpu.sync_copy(dst_hbm.at[c0 + b], dst_b[b])
            pltpu.async_copy(h_hbm.at[src_b[b]], rows[b], gsems[b])
        pltpu.async_copy(src_hbm.at[c0 + 2], src_b[2], isems[2])
        pltpu.async_copy(dst_hbm.at[c0 + 2], dst_b[2], isems[2])

        def zero_row(r, carry):
            for j in range(D // 16):
                rows[2][r, pl.ds(j * 16, 16)] = jnp.zeros((16,), jnp.float32)
            return carry
        lax.fori_loop(0, _C, zero_row, 0)
        base = sid * rows_per_tile
        nfull = rows_per_tile // _C
        for i in range(nfull):
            pltpu.sync_copy(rows[2], agg_sh.at[pl.ds(base + i * _C, _C)])
        rem = rows_per_tile - nfull * _C
        if rem:
            pltpu.sync_copy(rows[2].at[pl.ds(0, rem)],
                            agg_sh.at[pl.ds(base + nfull * _C, rem)])
        tail = rem_rows + _PAD_ROWS

        @pl.when(sid == _NS - 1)
        def _zero_tail():
            pltpu.sync_copy(rows[2].at[pl.ds(0, tail)],
                            agg_sh.at[pl.ds(rows_per_tile * _NS, tail)])
        plsc.subcore_barrier()

        def step(g, s):
            s2 = (s + 2) % 3

            @pl.when(g + 2 < cw)
            def _advance():
                pltpu.make_async_copy(src_hbm.at[c0 + g + 2], src_b[s2],
                                      isems[s2]).wait()
                pltpu.make_async_copy(dst_hbm.at[c0 + g + 2], dst_b[s2],
                                      isems[s2]).wait()

                @pl.when(g >= 1)
                def _drain_scatter():
                    pltpu.make_async_copy(rows[s2], agg_sh.at[dstv[s2]],
                                          ssems[s2]).wait()

                pltpu.async_copy(h_hbm.at[src_b[s2]], rows[s2], gsems[s2])

            pltpu.make_async_copy(h_hbm.at[src_b[s]], rows[s],
                                  gsems[s]).wait()
            for j in range(_C // 16):
                dstv[s][pl.ds(j * 16, 16)] = dst_b[s][pl.ds(j * 16, 16)]

            @pl.when(g + 3 < cw)
            def _prefetch_idx():
                pltpu.async_copy(src_hbm.at[c0 + g + 3], src_b[s], isems[s])
                pltpu.async_copy(dst_hbm.at[c0 + g + 3], dst_b[s], isems[s])

            pltpu.async_copy(rows[s], agg_sh.at[dstv[s]], ssems[s], add=True)

        def outer(go, carry):
            step(go * 3, 0)
            step(go * 3 + 1, 1)
            step(go * 3 + 2, 2)
            return carry
        lax.fori_loop(0, cw // 3, outer, 0)
        for s in ((cw - 3) % 3, (cw - 2) % 3, (cw - 1) % 3):
            pltpu.make_async_copy(rows[s], agg_sh.at[dstv[s]],
                                  ssems[s]).wait()
        plsc.subcore_barrier()

        pltpu.sync_copy(agg_sh.at[pl.ds(base, rows_per_tile)],
                        out_hbm.at[cid, pl.ds(base, rows_per_tile)])
        if rem_rows:
            @pl.when(sid == _NS - 1)
            def _write_tail():
                pltpu.sync_copy(
                    agg_sh.at[pl.ds(rows_per_tile * _NS, rem_rows)],
                    out_hbm.at[cid, pl.ds(rows_per_tile * _NS, rem_rows)])

    return agg_kernel(h, src2d, dst2d)


def _pad_edges(edge_index, N):
    E = edge_index.shape[1]
    step = _C * _NC * _NS * 3
    e_pad = -E % step
    src = edge_index[0]
    dst = edge_index[1]
    if e_pad:
        pad_ar = jnp.arange(e_pad, dtype=jnp.int32)
        src = jnp.concatenate([src, pad_ar % N])
        dst = jnp.concatenate([dst, N + (pad_ar % _PAD_ROWS)])
    nchunks = (E + e_pad) // _C
    return src.reshape(nchunks, _C), dst.reshape(nchunks, _C)


def _mlp_layer(h, p0, p1, w1, b1, w2, b2, g, bt):
    N, D = h.shape
    inv = 1.0 / math.sqrt(1.0 + 1e-5)

    def body(h_ref, p0_ref, p1_ref, w1_ref, b1_ref, w2_ref, b2_ref, g_ref,
             bt_ref, o_ref):
        hb = h_ref[...]
        z = hb + p0_ref[...] + p1_ref[...]
        z1 = jnp.dot(z, w1_ref[...], preferred_element_type=jnp.float32)
        z1 = jnp.maximum(z1 + b1_ref[...], 0.0)
        z2 = jnp.dot(z1, w2_ref[...], preferred_element_type=jnp.float32)
        z2 = z2 + b2_ref[...]
        z3 = g_ref[...] * (z2 * inv) + bt_ref[...]
        o_ref[...] = jnp.maximum(z3, 0.0) + hb

    full = pl.BlockSpec((1, D), lambda i: (0, 0))
    return pl.pallas_call(
        body,
        grid=(N // _BN,),
        in_specs=[
            pl.BlockSpec((_BN, D), lambda i: (i, 0)),
            pl.BlockSpec((_BN, D), lambda i: (i, 0)),
            pl.BlockSpec((_BN, D), lambda i: (i, 0)),
            pl.BlockSpec((D, D), lambda i: (0, 0)),
            full,
            pl.BlockSpec((D, D), lambda i: (0, 0)),
            full, full, full,
        ],
        out_specs=pl.BlockSpec((_BN, D), lambda i: (i, 0)),
        out_shape=jax.ShapeDtypeStruct((N, D), jnp.float32),
    )(h, p0, p1, w1, b1, w2, b2, g, bt)


def _mlp_pool_layer(h, p0, p1, w1, b1, w2, b2, g, bt, bv2d):
    N, D = h.shape
    inv = 1.0 / math.sqrt(1.0 + 1e-5)

    def body(h_ref, p0_ref, p1_ref, w1_ref, b1_ref, w2_ref, b2_ref, g_ref,
             bt_ref, bv_ref, o_ref, emb_ref, sums, cnts):
        i = pl.program_id(0)

        @pl.when(i == 0)
        def _init():
            sums[...] = jnp.zeros_like(sums)
            cnts[...] = jnp.zeros_like(cnts)

        hb = h_ref[...]
        z = hb + p0_ref[...] + p1_ref[...]
        z1 = jnp.dot(z, w1_ref[...], preferred_element_type=jnp.float32)
        z1 = jnp.maximum(z1 + b1_ref[...], 0.0)
        z2 = jnp.dot(z1, w2_ref[...], preferred_element_type=jnp.float32)
        z2 = z2 + b2_ref[...]
        z3 = g_ref[...] * (z2 * inv) + bt_ref[...]
        ho = jnp.maximum(z3, 0.0) + hb
        o_ref[...] = ho

        oh = (bv_ref[...] == lax.broadcasted_iota(jnp.int32, (1, _G), 1))
        oh = oh.astype(jnp.float32)
        dn = (((0,), (0,)), ((), ()))
        sums[...] += lax.dot_general(oh, ho, dn,
                                     preferred_element_type=jnp.float32)
        cnts[...] += lax.dot_general(oh, jnp.ones((_BN, D), jnp.float32), dn,
                                     preferred_element_type=jnp.float32)

        @pl.when(i == pl.num_programs(0) - 1)
        def _fin():
            emb_ref[...] = sums[...] / jnp.maximum(cnts[...], 1.0)

    full = pl.BlockSpec((1, D), lambda i: (0, 0))
    return pl.pallas_call(
        body,
        grid=(N // _BN,),
        in_specs=[
            pl.BlockSpec((_BN, D), lambda i: (i, 0)),
            pl.BlockSpec((_BN, D), lambda i: (i, 0)),
            pl.BlockSpec((_BN, D), lambda i: (i, 0)),
            pl.BlockSpec((D, D), lambda i: (0, 0)),
            full,
            pl.BlockSpec((D, D), lambda i: (0, 0)),
            full, full, full,
            pl.BlockSpec((_BN, 1), lambda i: (i, 0)),
        ],
        out_specs=[
            pl.BlockSpec((_BN, D), lambda i: (i, 0)),
            pl.BlockSpec((_G, D), lambda i: (0, 0)),
        ],
        out_shape=[
            jax.ShapeDtypeStruct((N, D), jnp.float32),
            jax.ShapeDtypeStruct((_G, D), jnp.float32),
        ],
        scratch_shapes=[
            pltpu.VMEM((_G, D), jnp.float32),
            pltpu.VMEM((_G, D), jnp.float32),
        ],
    )(h, p0, p1, w1, b1, w2, b2, g, bt, bv2d)


def kernel(x, W1, b1, W2, b2, gamma, beta, edge_index, batch_vec):
    N, D = x.shape
    L = W1.shape[0]
    E = edge_index.shape[1]
    src2d, dst2d = _pad_edges(edge_index, N)
    bv2d = batch_vec.reshape(N, 1)
    h = x
    for l in range(L - 1):
        parts = _sc_aggregate(h, src2d, dst2d)
        h = _mlp_layer(h, parts[0], parts[1], W1[l],
                       b1[l].reshape(1, D), W2[l], b2[l].reshape(1, D),
                       gamma[l].reshape(1, D), beta[l].reshape(1, D))
    l = L - 1
    parts = _sc_aggregate(h, src2d, dst2d)
    h, graph_emb = _mlp_pool_layer(
        h, parts[0], parts[1], W1[l], b1[l].reshape(1, D), W2[l],
        b2[l].reshape(1, D), gamma[l].reshape(1, D), beta[l].reshape(1, D),
        bv2d)
    return (h, graph_emb)


# speedup vs baseline: 1.0221x; 1.0221x over previous
import functools
import math

import jax
import jax.numpy as jnp
from jax import lax
from jax.experimental import pallas as pl
from jax.experimental.pallas import tpu as pltpu
from jax.experimental.pallas import tpu_sc as plsc

_NC = 2
_NS = 16
_C = 128
_G = 64
_BN = 2000
_PAD_ROWS = 16


def _sc_aggregate(h, src2d, dst2d):
    N, D = h.shape
    nchunks = src2d.shape[0]
    NW = _NC * _NS
    cw = nchunks // NW
    rows_per_tile = (N // _NS) // 8 * 8
    rem_rows = N - rows_per_tile * _NS
    n_acc = N + _PAD_ROWS
    mesh = plsc.VectorSubcoreMesh(core_axis_name="c", subcore_axis_name="s",
                                  num_cores=_NC, num_subcores=_NS)

    @functools.partial(
        pl.kernel,
        out_type=jax.ShapeDtypeStruct((_NC, N, D), jnp.float32),
        mesh=mesh,
        scratch_types=[
            [pltpu.VMEM((_C,), jnp.int32) for _ in range(3)],
            [pltpu.VMEM((_C,), jnp.int32) for _ in range(3)],
            [pltpu.VMEM((_C,), jnp.int32) for _ in range(3)],
            [pltpu.VMEM((_C, D), jnp.float32) for _ in range(3)],
            pltpu.VMEM_SHARED((n_acc, D), jnp.float32),
            [pltpu.SemaphoreType.DMA for _ in range(3)],
            [pltpu.SemaphoreType.DMA for _ in range(3)],
            [pltpu.SemaphoreType.DMA for _ in range(3)],
        ],
    )
    def agg_kernel(h_hbm, src_hbm, dst_hbm, out_hbm, src_b, dst_b, dstv,
                   rows, agg_sh, isems, gsems, ssems):
        cid = lax.axis_index("c")
        sid = lax.axis_index("s")
        w = sid * _NC + cid
        c0 = w * cw

        for b in range(2):
            pltpu.sync_copy(src_hbm.at[c0 + b], src_b[b])
            pltpu.sync_copy(dst_hbm.at[c0 + b], dst_b[b])
            pltpu.async_copy(h_hbm.at[src_b[b]], rows[b], gsems[b])
        pltpu.async_copy(src_hbm.at[c0 + 2], src_b[2], isems[2])
        pltpu.async_copy(dst_hbm.at[c0 + 2], dst_b[2], isems[2])

        def zero_row(r, carry):
            for j in range(D // 16):
                rows[2][r, pl.ds(j * 16, 16)] = jnp.zeros((16,), jnp.float32)
            return carry
        lax.fori_loop(0, _C, zero_row, 0)
        base = sid * rows_per_tile
        nfull = rows_per_tile // _C
        for i in range(nfull):
            pltpu.sync_copy(rows[2], agg_sh.at[pl.ds(base + i * _C, _C)])
        rem = rows_per_tile - nfull * _C
        if rem:
            pltpu.sync_copy(rows[2].at[pl.ds(0, rem)],
                            agg_sh.at[pl.ds(base + nfull * _C, rem)])
        tail = rem_rows + _PAD_ROWS

        @pl.when(sid == _NS - 1)
        def _zero_tail():
            pltpu.sync_copy(rows[2].at[pl.ds(0, tail)],
                            agg_sh.at[pl.ds(rows_per_tile * _NS, tail)])
        plsc.subcore_barrier()

        def step(g, s):
            s2 = (s + 2) % 3

            @pl.when(g + 2 < cw)
            def _advance():
                pltpu.make_async_copy(src_hbm.at[c0 + g + 2], src_b[s2],
                                      isems[s2]).wait()
                pltpu.make_async_copy(dst_hbm.at[c0 + g + 2], dst_b[s2],
                                      isems[s2]).wait()

                @pl.when(g >= 1)
                def _drain_scatter():
                    pltpu.make_async_copy(rows[s2], agg_sh.at[dstv[s2]],
                                          ssems[s2]).wait()

                pltpu.async_copy(h_hbm.at[src_b[s2]], rows[s2], gsems[s2])

            pltpu.make_async_copy(h_hbm.at[src_b[s]], rows[s],
                                  gsems[s]).wait()
            for j in range(_C // 16):
                dstv[s][pl.ds(j * 16, 16)] = dst_b[s][pl.ds(j * 16, 16)]

            @pl.when(g + 3 < cw)
            def _prefetch_idx():
                pltpu.async_copy(src_hbm.at[c0 + g + 3], src_b[s], isems[s])
                pltpu.async_copy(dst_hbm.at[c0 + g + 3], dst_b[s], isems[s])

            pltpu.async_copy(rows[s], agg_sh.at[dstv[s]], ssems[s], add=True)

        def outer(go, carry):
            step(go * 3, 0)
            step(go * 3 + 1, 1)
            step(go * 3 + 2, 2)
            return carry
        lax.fori_loop(0, cw // 3, outer, 0)
        for s in ((cw - 3) % 3, (cw - 2) % 3, (cw - 1) % 3):
            pltpu.make_async_copy(rows[s], agg_sh.at[dstv[s]],
                                  ssems[s]).wait()
        plsc.subcore_barrier()

        pltpu.sync_copy(agg_sh.at[pl.ds(base, rows_per_tile)],
                        out_hbm.at[cid, pl.ds(base, rows_per_tile)])
        if rem_rows:
            @pl.when(sid == _NS - 1)
            def _write_tail():
                pltpu.sync_copy(
                    agg_sh.at[pl.ds(rows_per_tile * _NS, rem_rows)],
                    out_hbm.at[cid, pl.ds(rows_per_tile * _NS, rem_rows)])

    return agg_kernel(h, src2d, dst2d)


def _pad_edges(edge_index, N):
    E = edge_index.shape[1]
    step = _C * _NC * _NS * 3
    e_pad = -E % step
    src = edge_index[0]
    dst = edge_index[1]
    if e_pad:
        pad_ar = jnp.arange(e_pad, dtype=jnp.int32)
        src = jnp.concatenate([src, pad_ar % N])
        dst = jnp.concatenate([dst, N + (pad_ar % _PAD_ROWS)])
    nchunks = (E + e_pad) // _C
    return src.reshape(nchunks, _C), dst.reshape(nchunks, _C)


def _mlp_layer(h, p0, p1, w1, b1, w2, b2, g, bt):
    N, D = h.shape
    inv = 1.0 / math.sqrt(1.0 + 1e-5)

    def body(h_ref, p0_ref, p1_ref, w1_ref, b1_ref, w2_ref, b2_ref, g_ref,
             bt_ref, o_ref):
        hb = h_ref[...]
        z = hb + p0_ref[...] + p1_ref[...]
        z1 = jnp.dot(z, w1_ref[...], preferred_element_type=jnp.float32)
        z1 = jnp.maximum(z1 + b1_ref[...], 0.0)
        z2 = jnp.dot(z1, w2_ref[...], preferred_element_type=jnp.float32)
        z2 = z2 + b2_ref[...]
        z3 = g_ref[...] * (z2 * inv) + bt_ref[...]
        o_ref[...] = jnp.maximum(z3, 0.0) + hb

    full = pl.BlockSpec((1, D), lambda i: (0, 0))
    return pl.pallas_call(
        body,
        grid=(N // _BN,),
        in_specs=[
            pl.BlockSpec((_BN, D), lambda i: (i, 0)),
            pl.BlockSpec((_BN, D), lambda i: (i, 0)),
            pl.BlockSpec((_BN, D), lambda i: (i, 0)),
            pl.BlockSpec((D, D), lambda i: (0, 0)),
            full,
            pl.BlockSpec((D, D), lambda i: (0, 0)),
            full, full, full,
        ],
        out_specs=pl.BlockSpec((_BN, D), lambda i: (i, 0)),
        out_shape=jax.ShapeDtypeStruct((N, D), jnp.float32),
    )(h, p0, p1, w1, b1, w2, b2, g, bt)


def _mlp_pool_layer(h, p0, p1, w1, b1, w2, b2, g, bt, bv2d):
    N, D = h.shape
    inv = 1.0 / math.sqrt(1.0 + 1e-5)

    def body(h_ref, p0_ref, p1_ref, w1_ref, b1_ref, w2_ref, b2_ref, g_ref,
             bt_ref, bv_ref, o_ref, emb_ref, sums, cnts):
        i = pl.program_id(0)

        @pl.when(i == 0)
        def _init():
            sums[...] = jnp.zeros_like(sums)
            cnts[...] = jnp.zeros_like(cnts)

        hb = h_ref[...]
        z = hb + p0_ref[...] + p1_ref[...]
        z1 = jnp.dot(z, w1_ref[...], preferred_element_type=jnp.float32)
        z1 = jnp.maximum(z1 + b1_ref[...], 0.0)
        z2 = jnp.dot(z1, w2_ref[...], preferred_element_type=jnp.float32)
        z2 = z2 + b2_ref[...]
        z3 = g_ref[...] * (z2 * inv) + bt_ref[...]
        ho = jnp.maximum(z3, 0.0) + hb
        o_ref[...] = ho

        oh = (bv_ref[...] == lax.broadcasted_iota(jnp.int32, (1, _G), 1))
        oh = oh.astype(jnp.float32)
        dn = (((0,), (0,)), ((), ()))
        sums[...] += lax.dot_general(oh, ho, dn,
                                     preferred_element_type=jnp.float32)
        cnts[...] += lax.dot_general(oh, jnp.ones((_BN, D), jnp.float32), dn,
                                     preferred_element_type=jnp.float32)

        @pl.when(i == pl.num_programs(0) - 1)
        def _fin():
            emb_ref[...] = sums[...] / jnp.maximum(cnts[...], 1.0)

    full = pl.BlockSpec((1, D), lambda i: (0, 0))
    return pl.pallas_call(
        body,
        grid=(N // _BN,),
        in_specs=[
            pl.BlockSpec((_BN, D), lambda i: (i, 0)),
            pl.BlockSpec((_BN, D), lambda i: (i, 0)),
            pl.BlockSpec((_BN, D), lambda i: (i, 0)),
            pl.BlockSpec((D, D), lambda i: (0, 0)),
            full,
            pl.BlockSpec((D, D), lambda i: (0, 0)),
            full, full, full,
            pl.BlockSpec((_BN, 1), lambda i: (i, 0)),
        ],
        out_specs=[
            pl.BlockSpec((_BN, D), lambda i: (i, 0)),
            pl.BlockSpec((_G, D), lambda i: (0, 0)),
        ],
        out_shape=[
            jax.ShapeDtypeStruct((N, D), jnp.float32),
            jax.ShapeDtypeStruct((_G, D), jnp.float32),
        ],
        scratch_shapes=[
            pltpu.VMEM((_G, D), jnp.float32),
            pltpu.VMEM((_G, D), jnp.float32),
        ],
    )(h, p0, p1, w1, b1, w2, b2, g, bt, bv2d)


def kernel(x, W1, b1, W2, b2, gamma, beta, edge_index, batch_vec):
    N, D = x.shape
    L = W1.shape[0]
    E = edge_index.shape[1]
    src2d, dst2d = _pad_edges(edge_index, N)
    bv2d = batch_vec.reshape(N, 1)
    h = x
    for l in range(L - 1):
        parts = _sc_aggregate(h, src2d, dst2d)
        h = _mlp_layer(h, parts[0], parts[1], W1[l],
                       b1[l].reshape(1, D), W2[l], b2[l].reshape(1, D),
                       gamma[l].reshape(1, D), beta[l].reshape(1, D))
    l = L - 1
    parts = _sc_aggregate(h, src2d, dst2d)
    h, graph_emb = _mlp_pool_layer(
        h, parts[0], parts[1], W1[l], b1[l].reshape(1, D), W2[l],
        b2[l].reshape(1, D), gamma[l].reshape(1, D), beta[l].reshape(1, D),
        bv2d)
    return (h, graph_emb)


# speedup vs baseline: 1.0468x; 1.0242x over previous
import functools
import math

import jax
import jax.numpy as jnp
from jax import lax
from jax.experimental import pallas as pl
from jax.experimental.pallas import tpu as pltpu
from jax.experimental.pallas import tpu_sc as plsc

_NC = 2
_NS = 16
_C = 128
_G = 64
_BN = 2000
_PAD_ROWS = 16


def _sc_aggregate(h, src2d, dst2d):
    N, D = h.shape
    nchunks = src2d.shape[0]
    NW = _NC * _NS
    cw = nchunks // NW
    rows_per_tile = (N // _NS) // 8 * 8
    rem_rows = N - rows_per_tile * _NS
    n_acc = N + _PAD_ROWS
    mesh = plsc.VectorSubcoreMesh(core_axis_name="c", subcore_axis_name="s",
                                  num_cores=_NC, num_subcores=_NS)

    @functools.partial(
        pl.kernel,
        out_type=jax.ShapeDtypeStruct((_NC, N, D), jnp.float32),
        mesh=mesh,
        scratch_types=[
            [pltpu.VMEM((_C,), jnp.int32) for _ in range(3)],
            [pltpu.VMEM((_C,), jnp.int32) for _ in range(3)],
            [pltpu.VMEM((_C,), jnp.int32) for _ in range(3)],
            [pltpu.VMEM((_C, D), jnp.float32) for _ in range(3)],
            pltpu.VMEM_SHARED((n_acc, D), jnp.float32),
            [pltpu.SemaphoreType.DMA for _ in range(3)],
            [pltpu.SemaphoreType.DMA for _ in range(3)],
            [pltpu.SemaphoreType.DMA for _ in range(3)],
        ],
    )
    def agg_kernel(h_hbm, src_hbm, dst_hbm, out_hbm, src_b, dst_b, dstv,
                   rows, agg_sh, isems, gsems, ssems):
        cid = lax.axis_index("c")
        sid = lax.axis_index("s")
        w = sid * _NC + cid
        c0 = w * cw

        for b in range(2):
            pltpu.sync_copy(src_hbm.at[c0 + b], src_b[b])
            pltpu.sync_copy(dst_hbm.at[c0 + b], dst_b[b])
            pltpu.async_copy(h_hbm.at[src_b[b]], rows[b], gsems[b])
        pltpu.async_copy(src_hbm.at[c0 + 2], src_b[2], isems[2])
        pltpu.async_copy(dst_hbm.at[c0 + 2], dst_b[2], isems[2])

        def zero_row(r, carry):
            for j in range(D // 16):
                rows[2][r, pl.ds(j * 16, 16)] = jnp.zeros((16,), jnp.float32)
            return carry
        lax.fori_loop(0, _C, zero_row, 0)
        base = sid * rows_per_tile
        nfull = rows_per_tile // _C
        for i in range(nfull):
            pltpu.sync_copy(rows[2], agg_sh.at[pl.ds(base + i * _C, _C)])
        rem = rows_per_tile - nfull * _C
        if rem:
            pltpu.sync_copy(rows[2].at[pl.ds(0, rem)],
                            agg_sh.at[pl.ds(base + nfull * _C, rem)])
        tail = rem_rows + _PAD_ROWS

        @pl.when(sid == _NS - 1)
        def _zero_tail():
            pltpu.sync_copy(rows[2].at[pl.ds(0, tail)],
                            agg_sh.at[pl.ds(rows_per_tile * _NS, tail)])
        plsc.subcore_barrier()

        def _when(cond, fn):
            if isinstance(cond, bool):
                if cond:
                    fn()
            else:
                pl.when(cond)(fn)

        def step(g, s):
            s2 = (s + 2) % 3

            def _advance():
                pltpu.make_async_copy(src_hbm.at[c0 + g + 2], src_b[s2],
                                      isems[s2]).wait()
                pltpu.make_async_copy(dst_hbm.at[c0 + g + 2], dst_b[s2],
                                      isems[s2]).wait()

                def _drain_scatter():
                    pltpu.make_async_copy(rows[s2], agg_sh.at[dstv[s2]],
                                          ssems[s2]).wait()
                _when(g >= 1, _drain_scatter)

                pltpu.async_copy(h_hbm.at[src_b[s2]], rows[s2], gsems[s2])
            _when(g + 2 < cw, _advance)

            pltpu.make_async_copy(h_hbm.at[src_b[s]], rows[s],
                                  gsems[s]).wait()
            for j in range(_C // 16):
                dstv[s][pl.ds(j * 16, 16)] = dst_b[s][pl.ds(j * 16, 16)]

            def _prefetch_idx():
                pltpu.async_copy(src_hbm.at[c0 + g + 3], src_b[s], isems[s])
                pltpu.async_copy(dst_hbm.at[c0 + g + 3], dst_b[s], isems[s])
            _when(g + 3 < cw, _prefetch_idx)

            pltpu.async_copy(rows[s], agg_sh.at[dstv[s]], ssems[s], add=True)

        def outer(go, carry):
            step(go * 3, 0)
            step(go * 3 + 1, 1)
            step(go * 3 + 2, 2)
            return carry
        lax.fori_loop(0, cw // 3, outer, 0)
        for k in range(cw - (cw // 3) * 3):
            step((cw // 3) * 3 + k, ((cw // 3) * 3 + k) % 3)
        for s in ((cw - 3) % 3, (cw - 2) % 3, (cw - 1) % 3):
            pltpu.make_async_copy(rows[s], agg_sh.at[dstv[s]],
                                  ssems[s]).wait()
        plsc.subcore_barrier()

        pltpu.sync_copy(agg_sh.at[pl.ds(base, rows_per_tile)],
                        out_hbm.at[cid, pl.ds(base, rows_per_tile)])
        if rem_rows:
            @pl.when(sid == _NS - 1)
            def _write_tail():
                pltpu.sync_copy(
                    agg_sh.at[pl.ds(rows_per_tile * _NS, rem_rows)],
                    out_hbm.at[cid, pl.ds(rows_per_tile * _NS, rem_rows)])

    return agg_kernel(h, src2d, dst2d)


def _pad_edges(edge_index, N):
    E = edge_index.shape[1]
    step = _C * _NC * _NS
    e_pad = -E % step
    src = edge_index[0]
    dst = edge_index[1]
    if e_pad:
        pad_ar = jnp.arange(e_pad, dtype=jnp.int32)
        src = jnp.concatenate([src, pad_ar % N])
        dst = jnp.concatenate([dst, N + (pad_ar % _PAD_ROWS)])
    nchunks = (E + e_pad) // _C
    return src.reshape(nchunks, _C), dst.reshape(nchunks, _C)


def _mlp_layer(h, p0, p1, w1, b1, w2, b2, g, bt):
    N, D = h.shape
    inv = 1.0 / math.sqrt(1.0 + 1e-5)

    def body(h_ref, p0_ref, p1_ref, w1_ref, b1_ref, w2_ref, b2_ref, g_ref,
             bt_ref, o_ref):
        hb = h_ref[...]
        z = hb + p0_ref[...] + p1_ref[...]
        z1 = jnp.dot(z, w1_ref[...], preferred_element_type=jnp.float32)
        z1 = jnp.maximum(z1 + b1_ref[...], 0.0)
        z2 = jnp.dot(z1, w2_ref[...], preferred_element_type=jnp.float32)
        z2 = z2 + b2_ref[...]
        z3 = g_ref[...] * (z2 * inv) + bt_ref[...]
        o_ref[...] = jnp.maximum(z3, 0.0) + hb

    full = pl.BlockSpec((1, D), lambda i: (0, 0))
    return pl.pallas_call(
        body,
        grid=(N // _BN,),
        in_specs=[
            pl.BlockSpec((_BN, D), lambda i: (i, 0)),
            pl.BlockSpec((_BN, D), lambda i: (i, 0)),
            pl.BlockSpec((_BN, D), lambda i: (i, 0)),
            pl.BlockSpec((D, D), lambda i: (0, 0)),
            full,
            pl.BlockSpec((D, D), lambda i: (0, 0)),
            full, full, full,
        ],
        out_specs=pl.BlockSpec((_BN, D), lambda i: (i, 0)),
        out_shape=jax.ShapeDtypeStruct((N, D), jnp.float32),
    )(h, p0, p1, w1, b1, w2, b2, g, bt)


def _mlp_pool_layer(h, p0, p1, w1, b1, w2, b2, g, bt, bv2d):
    N, D = h.shape
    inv = 1.0 / math.sqrt(1.0 + 1e-5)

    def body(h_ref, p0_ref, p1_ref, w1_ref, b1_ref, w2_ref, b2_ref, g_ref,
             bt_ref, bv_ref, o_ref, emb_ref, sums, cnts):
        i = pl.program_id(0)

        @pl.when(i == 0)
        def _init():
            sums[...] = jnp.zeros_like(sums)
            cnts[...] = jnp.zeros_like(cnts)

        hb = h_ref[...]
        z = hb + p0_ref[...] + p1_ref[...]
        z1 = jnp.dot(z, w1_ref[...], preferred_element_type=jnp.float32)
        z1 = jnp.maximum(z1 + b1_ref[...], 0.0)
        z2 = jnp.dot(z1, w2_ref[...], preferred_element_type=jnp.float32)
        z2 = z2 + b2_ref[...]
        z3 = g_ref[...] * (z2 * inv) + bt_ref[...]
        ho = jnp.maximum(z3, 0.0) + hb
        o_ref[...] = ho

        oh = (bv_ref[...] == lax.broadcasted_iota(jnp.int32, (1, _G), 1))
        oh = oh.astype(jnp.float32)
        dn = (((0,), (0,)), ((), ()))
        sums[...] += lax.dot_general(oh, ho, dn,
                                     preferred_element_type=jnp.float32)
        cnts[...] += lax.dot_general(oh, jnp.ones((_BN, D), jnp.float32), dn,
                                     preferred_element_type=jnp.float32)

        @pl.when(i == pl.num_programs(0) - 1)
        def _fin():
            emb_ref[...] = sums[...] / jnp.maximum(cnts[...], 1.0)

    full = pl.BlockSpec((1, D), lambda i: (0, 0))
    return pl.pallas_call(
        body,
        grid=(N // _BN,),
        in_specs=[
            pl.BlockSpec((_BN, D), lambda i: (i, 0)),
            pl.BlockSpec((_BN, D), lambda i: (i, 0)),
            pl.BlockSpec((_BN, D), lambda i: (i, 0)),
            pl.BlockSpec((D, D), lambda i: (0, 0)),
            full,
            pl.BlockSpec((D, D), lambda i: (0, 0)),
            full, full, full,
            pl.BlockSpec((_BN, 1), lambda i: (i, 0)),
        ],
        out_specs=[
            pl.BlockSpec((_BN, D), lambda i: (i, 0)),
            pl.BlockSpec((_G, D), lambda i: (0, 0)),
        ],
        out_shape=[
            jax.ShapeDtypeStruct((N, D), jnp.float32),
            jax.ShapeDtypeStruct((_G, D), jnp.float32),
        ],
        scratch_shapes=[
            pltpu.VMEM((_G, D), jnp.float32),
            pltpu.VMEM((_G, D), jnp.float32),
        ],
    )(h, p0, p1, w1, b1, w2, b2, g, bt, bv2d)


def kernel(x, W1, b1, W2, b2, gamma, beta, edge_index, batch_vec):
    N, D = x.shape
    L = W1.shape[0]
    E = edge_index.shape[1]
    src2d, dst2d = _pad_edges(edge_index, N)
    bv2d = batch_vec.reshape(N, 1)
    h = x
    for l in range(L - 1):
        parts = _sc_aggregate(h, src2d, dst2d)
        h = _mlp_layer(h, parts[0], parts[1], W1[l],
                       b1[l].reshape(1, D), W2[l], b2[l].reshape(1, D),
                       gamma[l].reshape(1, D), beta[l].reshape(1, D))
    l = L - 1
    parts = _sc_aggregate(h, src2d, dst2d)
    h, graph_emb = _mlp_pool_layer(
        h, parts[0], parts[1], W1[l], b1[l].reshape(1, D), W2[l],
        b2[l].reshape(1, D), gamma[l].reshape(1, D), beta[l].reshape(1, D),
        bv2d)
    return (h, graph_emb)
